# baseline (speedup 1.0000x reference)
import functools
import math

import jax
import jax.numpy as jnp
from jax import lax
from jax.experimental import pallas as pl
from jax.experimental.pallas import tpu as pltpu

_VMEM_LIMIT_BYTES = 64 * 1024 * 1024
_SQRT_2_OVER_PI = math.sqrt(2.0 / math.pi)


def _gelu_tanh(x):
    return 0.5 * x * (1.0 + jnp.tanh(_SQRT_2_OVER_PI
                                     * (x + 0.044715 * x * x * x)))


def _fused_kernel(tok_ref, hid_a_ref, hid_b_ref, w_ref, b_ref, out_ref,
                  obuf_ref, osem, *, n_bins, n_batch):
    b_idx = pl.program_id(0)
    cur = lax.rem(b_idx, 2)
    s_half = hid_a_ref.shape[1]
    s_len = 2 * s_half

    def _store(slot, batch):
        return pltpu.make_async_copy(
            obuf_ref.at[slot], out_ref.at[batch], osem.at[slot])

    @pl.when(b_idx >= 2)
    def _():
        _store(cur, b_idx - 2).wait()

    sub_iota = lax.broadcasted_iota(jnp.int32, (n_batch, 1), 0)
    tok_row = jnp.sum(jnp.where(sub_iota == b_idx, tok_ref[...], 0),
                      axis=0, keepdims=True)

    iota_m = lax.broadcasted_iota(jnp.int32, (n_bins, s_len), 0)
    oh_mt = (iota_m == tok_row).astype(jnp.bfloat16)

    cell = (jnp.dot(oh_mt[:, :s_half], hid_a_ref[0].astype(jnp.bfloat16),
                    preferred_element_type=jnp.float32)
            + jnp.dot(oh_mt[:, s_half:], hid_b_ref[0].astype(jnp.bfloat16),
                      preferred_element_type=jnp.float32))

    proj = lax.dot_general(cell.astype(jnp.bfloat16),
                           w_ref[...].astype(jnp.bfloat16),
                           (((1,), (1,)), ((), ())),
                           preferred_element_type=jnp.float32) + b_ref[...][None, :]
    cell2 = _gelu_tanh(proj).astype(jnp.bfloat16)

    n_chunks = 4
    s_c = s_len // n_chunks
    for c in range(n_chunks):
        obuf_ref[cur, pl.ds(c * s_c, s_c)] = lax.dot_general(
            oh_mt[:, c * s_c:(c + 1) * s_c], cell2, (((0,), (0,)), ((), ())),
            preferred_element_type=jnp.float32)
        pltpu.make_async_copy(obuf_ref.at[cur, pl.ds(c * s_c, s_c)],
                              out_ref.at[b_idx, pl.ds(c * s_c, s_c)],
                              osem.at[cur]).start()

    @pl.when(b_idx == n_batch - 1)
    def _():
        @pl.when(n_batch >= 2)
        def _():
            _store(1 - cur, b_idx - 1).wait()
        _store(cur, b_idx).wait()


def kernel(hidden_states, token_type_ids, weight, bias):
    B, S, H = hidden_states.shape
    n_bins = 256
    S2 = S // 2

    return pl.pallas_call(
        functools.partial(_fused_kernel, n_bins=n_bins, n_batch=B),
        out_shape=jax.ShapeDtypeStruct((B, S, H), jnp.float32),
        grid=(B,),
        in_specs=[
            pl.BlockSpec((B, S), lambda b: (0, 0)),
            pl.BlockSpec((1, S2, H), lambda b: (b, 0, 0)),
            pl.BlockSpec((1, S2, H), lambda b: (b, 1, 0)),
            pl.BlockSpec((H, H), lambda b: (0, 0)),
            pl.BlockSpec((H,), lambda b: (0,)),
        ],
        out_specs=pl.BlockSpec(memory_space=pl.ANY),
        scratch_shapes=[pltpu.VMEM((2, S, H), jnp.float32),
                        pltpu.SemaphoreType.DMA((2,))],
        compiler_params=pltpu.CompilerParams(
            dimension_semantics=("arbitrary",),
            vmem_limit_bytes=_VMEM_LIMIT_BYTES),
    )(token_type_ids, hidden_states, hidden_states, weight, bias)


# speedup vs baseline: 1.0128x; 1.0128x over previous
import functools
import math

import jax
import jax.numpy as jnp
from jax import lax
from jax.experimental import pallas as pl
from jax.experimental.pallas import tpu as pltpu

_VMEM_LIMIT_BYTES = 64 * 1024 * 1024
_SQRT_2_OVER_PI = math.sqrt(2.0 / math.pi)


def _gelu_tanh(x):
    return 0.5 * x * (1.0 + jnp.tanh(_SQRT_2_OVER_PI
                                     * (x + 0.044715 * x * x * x)))


def _fused_kernel(tok_ref, hid_a_ref, hid_b_ref, w_ref, b_ref, out_ref,
                  obuf_ref, osem, *, n_bins, n_batch):
    b_idx = pl.program_id(0)
    cur = lax.rem(b_idx, 2)
    s_half = hid_a_ref.shape[1]
    s_len = 2 * s_half

    def _store(slot, batch):
        return pltpu.make_async_copy(
            obuf_ref.at[slot], out_ref.at[batch], osem.at[slot])

    @pl.when(b_idx >= 2)
    def _():
        _store(cur, b_idx - 2).wait()

    sub_iota = lax.broadcasted_iota(jnp.int32, (n_batch, 1), 0)
    tok_row = jnp.sum(jnp.where(sub_iota == b_idx, tok_ref[...], 0),
                      axis=0, keepdims=True)

    iota_m = lax.broadcasted_iota(jnp.int32, (n_bins, s_len), 0)
    oh_mt = (iota_m == tok_row).astype(jnp.bfloat16)

    cell = (jnp.dot(oh_mt[:, :s_half], hid_a_ref[0].astype(jnp.bfloat16),
                    preferred_element_type=jnp.float32)
            + jnp.dot(oh_mt[:, s_half:], hid_b_ref[0].astype(jnp.bfloat16),
                      preferred_element_type=jnp.float32))

    proj = lax.dot_general(cell.astype(jnp.bfloat16),
                           w_ref[...].astype(jnp.bfloat16),
                           (((1,), (1,)), ((), ())),
                           preferred_element_type=jnp.float32) + b_ref[...][None, :]
    cell2 = _gelu_tanh(proj).astype(jnp.bfloat16)

    obuf_ref[cur] = lax.dot_general(oh_mt, cell2, (((0,), (0,)), ((), ())),
                                    preferred_element_type=jnp.float32)
    _store(cur, b_idx).start()

    @pl.when(b_idx == n_batch - 1)
    def _():
        @pl.when(n_batch >= 2)
        def _():
            _store(1 - cur, b_idx - 1).wait()
        _store(cur, b_idx).wait()


def kernel(hidden_states, token_type_ids, weight, bias):
    B, S, H = hidden_states.shape
    n_bins = 256
    S2 = S // 2

    return pl.pallas_call(
        functools.partial(_fused_kernel, n_bins=n_bins, n_batch=B),
        out_shape=jax.ShapeDtypeStruct((B, S, H), jnp.float32),
        grid=(B,),
        in_specs=[
            pl.BlockSpec((B, S), lambda b: (0, 0)),
            pl.BlockSpec((1, S2, H), lambda b: (b, 0, 0)),
            pl.BlockSpec((1, S2, H), lambda b: (b, 1, 0)),
            pl.BlockSpec((H, H), lambda b: (0, 0)),
            pl.BlockSpec((H,), lambda b: (0,)),
        ],
        out_specs=pl.BlockSpec(memory_space=pl.ANY),
        scratch_shapes=[pltpu.VMEM((2, S, H), jnp.float32),
                        pltpu.SemaphoreType.DMA((2,))],
        compiler_params=pltpu.CompilerParams(
            dimension_semantics=("arbitrary",),
            vmem_limit_bytes=_VMEM_LIMIT_BYTES),
    )(token_type_ids, hidden_states, hidden_states, weight, bias)
